# baseline (speedup 1.0000x reference)
import jax
import jax.numpy as jnp
from jax.experimental import pallas as pl


def _add_kernel(x_ref, p_ref, o_ref):
    o_ref[...] = x_ref[...] + p_ref[...]


def kernel(x, pos_table):
    B, T, D = x.shape
    BB = 2
    TB = 1024
    grid = (T // TB, B // BB)
    return pl.pallas_call(
        _add_kernel,
        grid=grid,
        in_specs=[
            pl.BlockSpec((BB, TB, D), lambda t, b: (b, t, 0)),
            pl.BlockSpec((TB, D), lambda t, b: (t, 0)),
        ],
        out_specs=pl.BlockSpec((BB, TB, D), lambda t, b: (b, t, 0)),
        out_shape=jax.ShapeDtypeStruct((B, T, D), x.dtype),
    )(x, pos_table)


# speedup vs baseline: 1.0122x; 1.0122x over previous
import jax
import jax.numpy as jnp
from jax.experimental import pallas as pl
from jax.experimental.pallas import tpu as pltpu


def _add_kernel(x_hbm, p_hbm, o_hbm, xbuf, obuf, pbuf, xsem, psem, osem):
    nb = x_hbm.shape[0]

    pltpu.make_async_copy(p_hbm, pbuf, psem).start()
    pltpu.make_async_copy(x_hbm.at[0], xbuf.at[0], xsem.at[0]).start()
    pltpu.make_async_copy(x_hbm.at[1], xbuf.at[1], xsem.at[1]).start()
    pltpu.make_async_copy(p_hbm, pbuf, psem).wait()

    for i in range(nb):
        slot = i % 2
        pltpu.make_async_copy(x_hbm.at[i], xbuf.at[slot], xsem.at[slot]).wait()
        if i >= 2:
            pltpu.make_async_copy(
                obuf.at[slot], o_hbm.at[i - 2], osem.at[slot]
            ).wait()
        obuf[slot] = xbuf[slot] + pbuf[...]
        pltpu.make_async_copy(obuf.at[slot], o_hbm.at[i], osem.at[slot]).start()
        if i + 2 < nb:
            pltpu.make_async_copy(
                x_hbm.at[i + 2], xbuf.at[slot], xsem.at[slot]
            ).start()

    for i in range(max(nb - 2, 0), nb):
        slot = i % 2
        pltpu.make_async_copy(obuf.at[slot], o_hbm.at[i], osem.at[slot]).wait()


def kernel(x, pos_table):
    B, T, D = x.shape
    return pl.pallas_call(
        _add_kernel,
        in_specs=[
            pl.BlockSpec(memory_space=pl.ANY),
            pl.BlockSpec(memory_space=pl.ANY),
        ],
        out_specs=pl.BlockSpec(memory_space=pl.ANY),
        out_shape=jax.ShapeDtypeStruct((B, T, D), x.dtype),
        scratch_shapes=[
            pltpu.VMEM((2, T, D), x.dtype),
            pltpu.VMEM((2, T, D), x.dtype),
            pltpu.VMEM((T, D), x.dtype),
            pltpu.SemaphoreType.DMA((2,)),
            pltpu.SemaphoreType.DMA,
            pltpu.SemaphoreType.DMA((2,)),
        ],
    )(x, pos_table)


# speedup vs baseline: 1.0805x; 1.0674x over previous
import jax
import jax.numpy as jnp
from jax.experimental import pallas as pl
from jax.experimental.pallas import tpu as pltpu


def _add_kernel(x_hbm, p_hbm, o_hbm, xbuf, obuf, pbuf, xsem, psem, osem):
    nb = x_hbm.shape[0] // 2

    pltpu.make_async_copy(p_hbm, pbuf, psem).start()
    for i in range(nb):
        pltpu.make_async_copy(
            x_hbm.at[pl.ds(2 * i, 2)], xbuf.at[i], xsem.at[i]
        ).start()
    pltpu.make_async_copy(p_hbm, pbuf, psem).wait()

    for i in range(nb):
        pltpu.make_async_copy(
            x_hbm.at[pl.ds(2 * i, 2)], xbuf.at[i], xsem.at[i]
        ).wait()
        obuf[i] = xbuf[i] + pbuf[...]
        pltpu.make_async_copy(
            obuf.at[i], o_hbm.at[pl.ds(2 * i, 2)], osem.at[i]
        ).start()

    for i in range(nb):
        pltpu.make_async_copy(
            obuf.at[i], o_hbm.at[pl.ds(2 * i, 2)], osem.at[i]
        ).wait()


def kernel(x, pos_table):
    B, T, D = x.shape
    return pl.pallas_call(
        _add_kernel,
        in_specs=[
            pl.BlockSpec(memory_space=pl.ANY),
            pl.BlockSpec(memory_space=pl.ANY),
        ],
        out_specs=pl.BlockSpec(memory_space=pl.ANY),
        out_shape=jax.ShapeDtypeStruct((B, T, D), x.dtype),
        scratch_shapes=[
            pltpu.VMEM((B // 2, 2, T, D), x.dtype),
            pltpu.VMEM((B // 2, 2, T, D), x.dtype),
            pltpu.VMEM((T, D), x.dtype),
            pltpu.SemaphoreType.DMA((B // 2,)),
            pltpu.SemaphoreType.DMA,
            pltpu.SemaphoreType.DMA((B // 2,)),
        ],
    )(x, pos_table)
